# baseline (speedup 1.0000x reference)
import functools

import jax
import jax.numpy as jnp
from jax import lax
from jax.experimental import pallas as pl
from jax.experimental.pallas import tpu as pltpu
from jax.experimental.pallas import tpu_sc as plsc

B = 128
S = 2048
D = 1024

_mesh = plsc.VectorSubcoreMesh(core_axis_name="c", subcore_axis_name="s",
                               num_cores=1)


@functools.partial(
    pl.kernel,
    mesh=_mesh,
    out_type=jax.ShapeDtypeStruct((2 * B, D), jnp.float32),
    scratch_types=[
        pltpu.VMEM((16,), jnp.int32),
        pltpu.VMEM((16,), jnp.int32),
        pltpu.VMEM((16, D), jnp.float32),
        pltpu.SemaphoreType.DMA,
    ],
)
def _gather_rows(table_hbm, pos1_hbm, pos2_hbm, out_hbm,
                 pos1_v, pos2_v, rows_v, sem):
    wid = lax.axis_index("s")
    par = wid >> 3
    g = wid & 7

    c1 = pltpu.async_copy(pos1_hbm.at[pl.ds(g * 16, 16)], pos1_v, sem)
    c2 = pltpu.async_copy(pos2_hbm.at[pl.ds(g * 16, 16)], pos2_v, sem)
    c1.wait()
    c2.wait()

    pv = jnp.where(par == 0, pos1_v[...], pos2_v[...])
    copies = []
    for k in range(16):
        row = (g * 16 + k) * S + pv[k]
        orow = ((g * 16 + k) << 1) + par
        copies.append(pltpu.async_copy(
            table_hbm.at[pl.ds(row, 1)], out_hbm.at[pl.ds(orow, 1)], sem))
    for c in copies:
        c.wait()


def kernel(base_encoding, pos1, pos2):
    table = base_encoding.reshape(B * S, D)
    out = _gather_rows(table, pos1.astype(jnp.int32), pos2.astype(jnp.int32))
    return out.reshape(B, 2 * D)


# speedup vs baseline: 2.3377x; 2.3377x over previous
import functools

import jax
import jax.numpy as jnp
from jax import lax
from jax.experimental import pallas as pl
from jax.experimental.pallas import tpu as pltpu
from jax.experimental.pallas import tpu_sc as plsc

B = 128
S = 2048
D = 1024

_mesh = plsc.VectorSubcoreMesh(core_axis_name="c", subcore_axis_name="s",
                               num_cores=1)


@functools.partial(
    pl.kernel,
    mesh=_mesh,
    out_type=jax.ShapeDtypeStruct((2 * B, D), jnp.float32),
    scratch_types=[
        pltpu.VMEM((16,), jnp.int32),
        pltpu.VMEM((16,), jnp.int32),
        pltpu.VMEM((16, D), jnp.float32),
        pltpu.SemaphoreType.DMA,
    ],
)
def _gather_rows(table_hbm, pos1_hbm, pos2_hbm, out_hbm,
                 pos1_v, pos2_v, rows_v, sem):
    wid = lax.axis_index("s")
    par = wid >> 3
    g = wid & 7

    c1 = pltpu.async_copy(pos1_hbm.at[pl.ds(g * 16, 16)], pos1_v, sem)
    c2 = pltpu.async_copy(pos2_hbm.at[pl.ds(g * 16, 16)], pos2_v, sem)
    c1.wait()
    c2.wait()

    j = lax.iota(jnp.int32, 16)
    bat = g * 16 + j
    idx = bat * S + jnp.where(par == 0, pos1_v[...], pos2_v[...])
    oidx = (bat << 1) + par
    pltpu.async_copy(table_hbm.at[idx], rows_v, sem).wait()
    pltpu.async_copy(rows_v, out_hbm.at[oidx], sem).wait()


def kernel(base_encoding, pos1, pos2):
    table = base_encoding.reshape(B * S, D)
    out = _gather_rows(table, pos1.astype(jnp.int32), pos2.astype(jnp.int32))
    return out.reshape(B, 2 * D)


# speedup vs baseline: 2.5633x; 1.0965x over previous
import functools

import jax
import jax.numpy as jnp
from jax import lax
from jax.experimental import pallas as pl
from jax.experimental.pallas import tpu as pltpu
from jax.experimental.pallas import tpu_sc as plsc

B = 128
S = 2048
D = 1024

_mesh = plsc.VectorSubcoreMesh(core_axis_name="c", subcore_axis_name="s",
                               num_cores=1)


@functools.partial(
    pl.kernel,
    mesh=_mesh,
    out_type=jax.ShapeDtypeStruct((B, 2 * D), jnp.float32),
    scratch_types=[
        pltpu.VMEM((16,), jnp.int32),
        pltpu.VMEM((16, D), jnp.float32),
        pltpu.SemaphoreType.DMA,
    ],
)
def _gather_rows(table_hbm, posi_hbm, out_hbm, posi_v, rows_v, sem):
    w = lax.axis_index("s")

    pltpu.async_copy(posi_hbm.at[pl.ds(w * 16, 16)], posi_v, sem).wait()

    j = lax.iota(jnp.int32, 16)
    bat = w * 8 + (j >> 1)
    idx = bat * S + posi_v[...]
    pltpu.async_copy(table_hbm.at[idx], rows_v, sem).wait()
    copies = [
        pltpu.async_copy(
            rows_v.at[pl.ds(m, 1)],
            out_hbm.at[pl.ds(w * 8 + (m >> 1), 1), pl.ds((m & 1) * D, D)],
            sem)
        for m in range(16)
    ]
    for c in copies:
        c.wait()


def kernel(base_encoding, pos1, pos2):
    table = base_encoding.reshape(B * S, D)
    posi = jnp.stack([pos1.astype(jnp.int32), pos2.astype(jnp.int32)],
                     axis=1).reshape(2 * B)
    return _gather_rows(table, posi)


# speedup vs baseline: 2.5755x; 1.0048x over previous
import functools

import jax
import jax.numpy as jnp
from jax import lax
from jax.experimental import pallas as pl
from jax.experimental.pallas import tpu as pltpu
from jax.experimental.pallas import tpu_sc as plsc

B = 128
S = 2048
D = 1024

_mesh = plsc.VectorSubcoreMesh(core_axis_name="c", subcore_axis_name="s",
                               num_cores=1)


@functools.partial(
    pl.kernel,
    mesh=_mesh,
    out_type=jax.ShapeDtypeStruct((B, 2 * D), jnp.float32),
    scratch_types=[
        pltpu.VMEM((16,), jnp.int32),
        pltpu.VMEM((16,), jnp.int32),
        pltpu.VMEM((16, D), jnp.float32),
        pltpu.SemaphoreType.DMA,
    ],
)
def _gather_rows(table_hbm, pos1_hbm, pos2_hbm, out_hbm,
                 pos1_v, pos2_v, rows_v, sem):
    wid = lax.axis_index("s")
    par = wid >> 3
    g = wid & 7

    c1 = pltpu.async_copy(pos1_hbm.at[pl.ds(g * 16, 16)], pos1_v, sem)
    c2 = pltpu.async_copy(pos2_hbm.at[pl.ds(g * 16, 16)], pos2_v, sem)
    c1.wait()
    c2.wait()

    j = lax.iota(jnp.int32, 16)
    bat = g * 16 + j
    idx = bat * S + jnp.where(par == 0, pos1_v[...], pos2_v[...])
    pltpu.async_copy(table_hbm.at[idx], rows_v, sem).wait()
    copies = [
        pltpu.async_copy(
            rows_v.at[pl.ds(m, 1)],
            out_hbm.at[pl.ds(g * 16 + m, 1), pl.ds(par * D, D)],
            sem)
        for m in range(16)
    ]
    for c in copies:
        c.wait()


def kernel(base_encoding, pos1, pos2):
    table = base_encoding.reshape(B * S, D)
    return _gather_rows(table, pos1.astype(jnp.int32), pos2.astype(jnp.int32))
